# baseline (speedup 1.0000x reference)
import functools

import jax
import jax.numpy as jnp
from jax import lax
from jax.experimental import pallas as pl
from jax.experimental.pallas import tpu as pltpu
from jax.experimental.pallas import tpu_sc as plsc

_N_NODES = 10000
_N_EDGES = 320000
_D = 128

_info = plsc.get_sparse_core_info()
_NC = _info.num_cores
_NS = _info.num_subcores
_L = _info.num_lanes
_NW = _NC * _NS
_E_PER_W = _N_EDGES // _NW


def _tc_tables_body(x_ref, w1_ref, w2_ref, b_ref, s_ref, t_ref):
    x = x_ref[...]
    s = jnp.dot(x, w1_ref[...], preferred_element_type=jnp.float32)
    t = jnp.dot(x, w2_ref[...], preferred_element_type=jnp.float32)
    s_ref[...] = s.reshape(s.shape[0]) + b_ref[0]
    t_ref[...] = t.reshape(t.shape[0])


def _make_tables(node_embedding, w1, w2, b):
    s, t = pl.pallas_call(
        _tc_tables_body,
        in_specs=[
            pl.BlockSpec((_N_NODES, _D), lambda: (0, 0)),
            pl.BlockSpec((_D, 1), lambda: (0, 0)),
            pl.BlockSpec((_D, 1), lambda: (0, 0)),
            pl.BlockSpec(memory_space=pltpu.SMEM),
        ],
        out_specs=[
            pl.BlockSpec((_N_NODES,), lambda: (0,)),
            pl.BlockSpec((_N_NODES,), lambda: (0,)),
        ],
        out_shape=[
            jax.ShapeDtypeStruct((_N_NODES,), jnp.float32),
            jax.ShapeDtypeStruct((_N_NODES,), jnp.float32),
        ],
    )(node_embedding, w1, w2, b)
    return s, t


_sc_mesh = plsc.VectorSubcoreMesh(core_axis_name="c", subcore_axis_name="s")


@functools.partial(
    pl.kernel,
    mesh=_sc_mesh,
    out_type=jax.ShapeDtypeStruct((_N_EDGES,), jnp.float32),
    compiler_params=pltpu.CompilerParams(
        needs_layout_passes=False, use_tc_tiling_on_sc=False
    ),
    scratch_types=[
        pltpu.VMEM((_N_NODES,), jnp.float32),
        pltpu.VMEM((_N_NODES,), jnp.float32),
        pltpu.VMEM((_E_PER_W, 2), jnp.int32),
        pltpu.VMEM((_E_PER_W,), jnp.float32),
        pltpu.SemaphoreType.DMA,
    ],
)
def _sc_edge_logits(s_hbm, t_hbm, edges_hbm, out_hbm, s_v, t_v, e_v, o_v, sem):
    wid = lax.axis_index("s") * _NC + lax.axis_index("c")
    base = wid * _E_PER_W
    c1 = pltpu.async_copy(s_hbm, s_v, sem)
    c2 = pltpu.async_copy(t_hbm, t_v, sem)
    c3 = pltpu.async_copy(edges_hbm.at[pl.ds(base, _E_PER_W), :], e_v, sem)
    c1.wait()
    c2.wait()
    c3.wait()

    lane = lax.iota(jnp.int32, _L)
    zeros = jnp.zeros((_L,), jnp.int32)
    ones = jnp.ones((_L,), jnp.int32)

    def body(i, carry):
        row = _L * i + lane
        src = plsc.load_gather(e_v, [row, zeros])
        dst = plsc.load_gather(e_v, [row, ones])
        gs = plsc.load_gather(s_v, [src])
        gt = plsc.load_gather(t_v, [dst])
        o_v[pl.ds(i * _L, _L)] = gs + gt
        return carry

    lax.fori_loop(0, _E_PER_W // _L, body, 0, unroll=4)
    pltpu.sync_copy(o_v, out_hbm.at[pl.ds(base, _E_PER_W)])


def kernel(node_embedding, edges, W, b):
    w1 = W[0, :_D].reshape(_D, 1)
    w2 = W[0, _D:].reshape(_D, 1)
    s, t = _make_tables(node_embedding, w1, w2, b)
    return _sc_edge_logits(s, t, edges.astype(jnp.int32)).reshape(_N_EDGES, 1)


# speedup vs baseline: 5.1380x; 5.1380x over previous
import functools

import jax
import jax.numpy as jnp
from jax import lax
from jax.experimental import pallas as pl
from jax.experimental.pallas import tpu as pltpu
from jax.experimental.pallas import tpu_sc as plsc

_N_NODES = 10000
_N_EDGES = 320000
_D = 128

_info = plsc.get_sparse_core_info()
_NC = _info.num_cores
_NS = _info.num_subcores
_L = _info.num_lanes
_NW = _NC * _NS
_E_PER_W = _N_EDGES // _NW


def _tc_tables_body(x_ref, w1_ref, w2_ref, b_ref, s_ref, t_ref):
    x = x_ref[...]
    s = jnp.dot(x, w1_ref[...], preferred_element_type=jnp.float32)
    t = jnp.dot(x, w2_ref[...], preferred_element_type=jnp.float32)
    s_ref[...] = s.reshape(s.shape[0]) + b_ref[0]
    t_ref[...] = t.reshape(t.shape[0])


def _make_tables(node_embedding, w1, w2, b):
    s, t = pl.pallas_call(
        _tc_tables_body,
        in_specs=[
            pl.BlockSpec((_N_NODES, _D), lambda: (0, 0)),
            pl.BlockSpec((_D, 1), lambda: (0, 0)),
            pl.BlockSpec((_D, 1), lambda: (0, 0)),
            pl.BlockSpec(memory_space=pltpu.SMEM),
        ],
        out_specs=[
            pl.BlockSpec((_N_NODES,), lambda: (0,)),
            pl.BlockSpec((_N_NODES,), lambda: (0,)),
        ],
        out_shape=[
            jax.ShapeDtypeStruct((_N_NODES,), jnp.float32),
            jax.ShapeDtypeStruct((_N_NODES,), jnp.float32),
        ],
    )(node_embedding, w1, w2, b)
    return s, t


_sc_mesh = plsc.VectorSubcoreMesh(core_axis_name="c", subcore_axis_name="s")


@functools.partial(
    pl.kernel,
    mesh=_sc_mesh,
    out_type=jax.ShapeDtypeStruct((_N_EDGES,), jnp.float32),
    compiler_params=pltpu.CompilerParams(needs_layout_passes=False),
    scratch_types=[
        pltpu.VMEM((_N_NODES,), jnp.float32),
        pltpu.VMEM((_N_NODES,), jnp.float32),
        pltpu.VMEM((_E_PER_W,), jnp.int32),
        pltpu.VMEM((_E_PER_W,), jnp.int32),
        pltpu.VMEM((_E_PER_W,), jnp.float32),
        pltpu.SemaphoreType.DMA,
        pltpu.SemaphoreType.DMA,
    ],
)
def _sc_edge_logits(s_hbm, t_hbm, src_hbm, dst_hbm, out_hbm,
                    s_v, t_v, src_v, dst_v, o_v, sem, osem):
    wid = lax.axis_index("s") * _NC + lax.axis_index("c")
    base = wid * _E_PER_W
    nchunks = 5
    cw = _E_PER_W // nchunks

    c1 = pltpu.async_copy(s_hbm, s_v, sem)
    c2 = pltpu.async_copy(t_hbm, t_v, sem)
    copies = []
    for k in range(nchunks):
        sl = pl.ds(base + k * cw, cw)
        vl = pl.ds(k * cw, cw)
        copies.append((
            pltpu.async_copy(src_hbm.at[sl], src_v.at[vl], sem),
            pltpu.async_copy(dst_hbm.at[sl], dst_v.at[vl], sem),
        ))
    c1.wait()
    c2.wait()

    def body(i, carry):
        sl = pl.ds(i * _L, _L)
        gs = plsc.load_gather(s_v, [src_v[sl]])
        gt = plsc.load_gather(t_v, [dst_v[sl]])
        o_v[sl] = gs + gt
        return carry

    it_per_chunk = cw // _L
    for k in range(nchunks):
        copies[k][0].wait()
        copies[k][1].wait()
        lax.fori_loop(k * it_per_chunk, (k + 1) * it_per_chunk, body, 0,
                      unroll=8)
        pltpu.async_copy(o_v.at[pl.ds(k * cw, cw)],
                         out_hbm.at[pl.ds(base + k * cw, cw)], osem)
    for k in range(nchunks):
        pltpu.make_async_copy(o_v.at[pl.ds(k * cw, cw)],
                              out_hbm.at[pl.ds(base + k * cw, cw)],
                              osem).wait()


def kernel(node_embedding, edges, W, b):
    w1 = W[0, :_D].reshape(_D, 1)
    w2 = W[0, _D:].reshape(_D, 1)
    src = edges[:, 0].astype(jnp.int32)
    dst = edges[:, 1].astype(jnp.int32)
    s, t = _make_tables(node_embedding, w1, w2, b)
    return _sc_edge_logits(s, t, src, dst).reshape(_N_EDGES, 1)


# speedup vs baseline: 5.2443x; 1.0207x over previous
import functools

import jax
import jax.numpy as jnp
from jax import lax
from jax.experimental import pallas as pl
from jax.experimental.pallas import tpu as pltpu
from jax.experimental.pallas import tpu_sc as plsc

_N_NODES = 10000
_N_EDGES = 320000
_D = 128

_info = plsc.get_sparse_core_info()
_NC = _info.num_cores
_NS = _info.num_subcores
_L = _info.num_lanes
_NW = _NC * _NS
_E_PER_W = _N_EDGES // _NW


_NBLK = 512
_N_PAD = 20 * _NBLK


def _tc_tables_body(x_ref, w12t_ref, b_ref, s_ref, t_ref):
    r = lax.dot_general(
        w12t_ref[...], x_ref[...],
        dimension_numbers=(((1,), (1,)), ((), ())),
        preferred_element_type=jnp.float32,
    )
    s_ref[...] = r[0:1, :].reshape(_NBLK) + b_ref[0]
    t_ref[...] = r[1:2, :].reshape(_NBLK)


def _make_tables(node_embedding, w12t, b):
    s, t = pl.pallas_call(
        _tc_tables_body,
        grid=(_N_PAD // _NBLK,),
        in_specs=[
            pl.BlockSpec((_NBLK, _D), lambda i: (i, 0)),
            pl.BlockSpec((2, _D), lambda i: (0, 0)),
            pl.BlockSpec(memory_space=pltpu.SMEM),
        ],
        out_specs=[
            pl.BlockSpec((_NBLK,), lambda i: (i,)),
            pl.BlockSpec((_NBLK,), lambda i: (i,)),
        ],
        out_shape=[
            jax.ShapeDtypeStruct((_N_PAD,), jnp.float32),
            jax.ShapeDtypeStruct((_N_PAD,), jnp.float32),
        ],
    )(node_embedding, w12t, b)
    return s, t


_sc_mesh = plsc.VectorSubcoreMesh(core_axis_name="c", subcore_axis_name="s")


@functools.partial(
    pl.kernel,
    mesh=_sc_mesh,
    out_type=jax.ShapeDtypeStruct((_N_EDGES,), jnp.float32),
    compiler_params=pltpu.CompilerParams(needs_layout_passes=False),
    scratch_types=[
        pltpu.VMEM((_N_PAD,), jnp.float32),
        pltpu.VMEM((_N_PAD,), jnp.float32),
        pltpu.VMEM((_E_PER_W,), jnp.int32),
        pltpu.VMEM((_E_PER_W,), jnp.int32),
        pltpu.VMEM((_E_PER_W,), jnp.float32),
        pltpu.SemaphoreType.DMA,
        pltpu.SemaphoreType.DMA,
    ],
)
def _sc_edge_logits(s_hbm, t_hbm, src_hbm, dst_hbm, out_hbm,
                    s_v, t_v, src_v, dst_v, o_v, sem, osem):
    wid = lax.axis_index("s") * _NC + lax.axis_index("c")
    base = wid * _E_PER_W
    nchunks = 5
    cw = _E_PER_W // nchunks

    c1 = pltpu.async_copy(s_hbm, s_v, sem)
    c2 = pltpu.async_copy(t_hbm, t_v, sem)
    copies = []
    for k in range(nchunks):
        sl = pl.ds(base + k * cw, cw)
        vl = pl.ds(k * cw, cw)
        copies.append((
            pltpu.async_copy(src_hbm.at[sl], src_v.at[vl], sem),
            pltpu.async_copy(dst_hbm.at[sl], dst_v.at[vl], sem),
        ))
    c1.wait()
    c2.wait()

    def body(i, carry):
        sl = pl.ds(i * _L, _L)
        gs = plsc.load_gather(s_v, [src_v[sl]])
        gt = plsc.load_gather(t_v, [dst_v[sl]])
        o_v[sl] = gs + gt
        return carry

    it_per_chunk = cw // _L
    for k in range(nchunks):
        copies[k][0].wait()
        copies[k][1].wait()
        lax.fori_loop(k * it_per_chunk, (k + 1) * it_per_chunk, body, 0,
                      unroll=8)
        pltpu.async_copy(o_v.at[pl.ds(k * cw, cw)],
                         out_hbm.at[pl.ds(base + k * cw, cw)], osem)
    for k in range(nchunks):
        pltpu.make_async_copy(o_v.at[pl.ds(k * cw, cw)],
                              out_hbm.at[pl.ds(base + k * cw, cw)],
                              osem).wait()


def kernel(node_embedding, edges, W, b):
    w12t = W.reshape(2, _D)
    src = edges[:, 0].astype(jnp.int32)
    dst = edges[:, 1].astype(jnp.int32)
    s, t = _make_tables(node_embedding, w12t, b)
    return _sc_edge_logits(s, t, src, dst).reshape(_N_EDGES, 1)


# speedup vs baseline: 5.6955x; 1.0860x over previous
import functools

import jax
import jax.numpy as jnp
from jax import lax
from jax.experimental import pallas as pl
from jax.experimental.pallas import tpu as pltpu
from jax.experimental.pallas import tpu_sc as plsc

_N_NODES = 10000
_N_EDGES = 320000
_D = 128

_info = plsc.get_sparse_core_info()
_NC = _info.num_cores
_NS = _info.num_subcores
_L = _info.num_lanes
_NW = _NC * _NS
_E_PER_W = _N_EDGES // _NW


_NBLK = 1024
_N_PAD = 10 * _NBLK


def _tc_tables_body(x_ref, w12t_ref, b_ref, s_ref, t_ref):
    r = lax.dot_general(
        w12t_ref[...], x_ref[...],
        dimension_numbers=(((1,), (1,)), ((), ())),
        preferred_element_type=jnp.float32,
    )
    s_ref[...] = r[0:1, :].reshape(_NBLK) + b_ref[0]
    t_ref[...] = r[1:2, :].reshape(_NBLK)


def _make_tables(node_embedding, w12t, b):
    s, t = pl.pallas_call(
        _tc_tables_body,
        grid=(_N_PAD // _NBLK,),
        in_specs=[
            pl.BlockSpec((_NBLK, _D), lambda i: (i, 0)),
            pl.BlockSpec((2, _D), lambda i: (0, 0)),
            pl.BlockSpec(memory_space=pltpu.SMEM),
        ],
        out_specs=[
            pl.BlockSpec((_NBLK,), lambda i: (i,)),
            pl.BlockSpec((_NBLK,), lambda i: (i,)),
        ],
        out_shape=[
            jax.ShapeDtypeStruct((_N_PAD,), jnp.float32),
            jax.ShapeDtypeStruct((_N_PAD,), jnp.float32),
        ],
    )(node_embedding, w12t, b)
    return s, t


_sc_mesh = plsc.VectorSubcoreMesh(core_axis_name="c", subcore_axis_name="s")


@functools.partial(
    pl.kernel,
    mesh=_sc_mesh,
    out_type=jax.ShapeDtypeStruct((_N_EDGES,), jnp.float32),
    compiler_params=pltpu.CompilerParams(needs_layout_passes=False),
    scratch_types=[
        pltpu.VMEM((_N_PAD,), jnp.float32),
        pltpu.VMEM((_N_PAD,), jnp.float32),
        pltpu.VMEM((_E_PER_W,), jnp.int32),
        pltpu.VMEM((_E_PER_W,), jnp.int32),
        pltpu.VMEM((_E_PER_W,), jnp.float32),
        pltpu.SemaphoreType.DMA,
        pltpu.SemaphoreType.DMA,
    ],
)
def _sc_edge_logits(s_hbm, t_hbm, src_hbm, dst_hbm, out_hbm,
                    s_v, t_v, src_v, dst_v, o_v, sem, osem):
    wid = lax.axis_index("s") * _NC + lax.axis_index("c")
    base = wid * _E_PER_W
    nchunks = 5
    cw = _E_PER_W // nchunks

    c1 = pltpu.async_copy(s_hbm, s_v, sem)
    c2 = pltpu.async_copy(t_hbm, t_v, sem)
    copies = []
    for k in range(nchunks):
        sl = pl.ds(base + k * cw, cw)
        vl = pl.ds(k * cw, cw)
        copies.append((
            pltpu.async_copy(src_hbm.at[sl], src_v.at[vl], sem),
            pltpu.async_copy(dst_hbm.at[sl], dst_v.at[vl], sem),
        ))
    c1.wait()
    c2.wait()

    def body(i, carry):
        sl = pl.ds(i * _L, _L)
        gs = plsc.load_gather(s_v, [src_v[sl]])
        gt = plsc.load_gather(t_v, [dst_v[sl]])
        o_v[sl] = gs + gt
        return carry

    it_per_chunk = cw // _L
    for k in range(nchunks):
        copies[k][0].wait()
        copies[k][1].wait()
        lax.fori_loop(k * it_per_chunk, (k + 1) * it_per_chunk, body, 0,
                      unroll=8)
        pltpu.async_copy(o_v.at[pl.ds(k * cw, cw)],
                         out_hbm.at[pl.ds(base + k * cw, cw)], osem)
    for k in range(nchunks):
        pltpu.make_async_copy(o_v.at[pl.ds(k * cw, cw)],
                              out_hbm.at[pl.ds(base + k * cw, cw)],
                              osem).wait()


def kernel(node_embedding, edges, W, b):
    w12t = W.reshape(2, _D)
    src = edges[:, 0].astype(jnp.int32)
    dst = edges[:, 1].astype(jnp.int32)
    s, t = _make_tables(node_embedding, w12t, b)
    return _sc_edge_logits(s, t, src, dst).reshape(_N_EDGES, 1)


# speedup vs baseline: 6.1491x; 1.0796x over previous
import functools

import jax
import jax.numpy as jnp
from jax import lax
from jax.experimental import pallas as pl
from jax.experimental.pallas import tpu as pltpu
from jax.experimental.pallas import tpu_sc as plsc

_N_NODES = 10000
_N_EDGES = 320000
_D = 128

_info = plsc.get_sparse_core_info()
_NC = _info.num_cores
_NS = _info.num_subcores
_L = _info.num_lanes
_NW = _NC * _NS
_E_PER_W = _N_EDGES // _NW


_NBLK = 1024
_N_PAD = 10 * _NBLK


def _tc_tables_body(x_ref, w12t_ref, b_ref, s_ref, t_ref):
    r = lax.dot_general(
        w12t_ref[...], x_ref[...],
        dimension_numbers=(((1,), (1,)), ((), ())),
        preferred_element_type=jnp.float32,
    )
    s_ref[...] = r[0:1, :].reshape(_NBLK) + b_ref[0]
    t_ref[...] = r[1:2, :].reshape(_NBLK)


def _make_tables(node_embedding, w12t, b):
    s, t = pl.pallas_call(
        _tc_tables_body,
        grid=(_N_PAD // _NBLK,),
        in_specs=[
            pl.BlockSpec((_NBLK, _D), lambda i: (i, 0)),
            pl.BlockSpec((2, _D), lambda i: (0, 0)),
            pl.BlockSpec(memory_space=pltpu.SMEM),
        ],
        out_specs=[
            pl.BlockSpec((_NBLK,), lambda i: (i,)),
            pl.BlockSpec((_NBLK,), lambda i: (i,)),
        ],
        out_shape=[
            jax.ShapeDtypeStruct((_N_PAD,), jnp.float32),
            jax.ShapeDtypeStruct((_N_PAD,), jnp.float32),
        ],
    )(node_embedding, w12t, b)
    return s, t


_sc_mesh = plsc.VectorSubcoreMesh(core_axis_name="c", subcore_axis_name="s")


@functools.partial(
    pl.kernel,
    mesh=_sc_mesh,
    out_type=jax.ShapeDtypeStruct((_N_EDGES,), jnp.float32),
    compiler_params=pltpu.CompilerParams(needs_layout_passes=False),
    scratch_types=[
        pltpu.VMEM((_N_PAD,), jnp.float32),
        pltpu.VMEM((_N_PAD,), jnp.float32),
        pltpu.VMEM((_E_PER_W,), jnp.int32),
        pltpu.VMEM((_E_PER_W,), jnp.int32),
        pltpu.VMEM((_E_PER_W,), jnp.float32),
        pltpu.SemaphoreType.DMA,
        pltpu.SemaphoreType.DMA,
    ],
)
def _sc_edge_logits(s_hbm, t_hbm, src_hbm, dst_hbm, out_hbm,
                    s_v, t_v, src_v, dst_v, o_v, sem, osem):
    wid = lax.axis_index("s") * _NC + lax.axis_index("c")
    base = wid * _E_PER_W
    nchunks = 5
    cw = _E_PER_W // nchunks

    c1 = pltpu.async_copy(s_hbm, s_v, sem)
    c2 = pltpu.async_copy(t_hbm, t_v, sem)
    copies = []
    for k in range(nchunks):
        sl = pl.ds(base + k * cw, cw)
        vl = pl.ds(k * cw, cw)
        copies.append((
            pltpu.async_copy(src_hbm.at[sl], src_v.at[vl], sem),
            pltpu.async_copy(dst_hbm.at[sl], dst_v.at[vl], sem),
        ))
    c1.wait()
    c2.wait()

    def body(i):
        sl = pl.ds(i, _L)
        gs = plsc.load_gather(s_v, [src_v[sl]])
        gt = plsc.load_gather(t_v, [dst_v[sl]])
        o_v[sl] = gs + gt

    for k in range(nchunks):
        copies[k][0].wait()
        copies[k][1].wait()
        plsc.parallel_loop(k * cw, (k + 1) * cw, _L, unroll=8)(body)
        pltpu.async_copy(o_v.at[pl.ds(k * cw, cw)],
                         out_hbm.at[pl.ds(base + k * cw, cw)], osem)
    for k in range(nchunks):
        pltpu.make_async_copy(o_v.at[pl.ds(k * cw, cw)],
                              out_hbm.at[pl.ds(base + k * cw, cw)],
                              osem).wait()


def kernel(node_embedding, edges, W, b):
    w12t = W.reshape(2, _D)
    src = edges[:, 0].astype(jnp.int32)
    dst = edges[:, 1].astype(jnp.int32)
    s, t = _make_tables(node_embedding, w12t, b)
    return _sc_edge_logits(s, t, src, dst).reshape(_N_EDGES, 1)
